# Initial kernel scaffold; baseline (speedup 1.0000x reference)
import jax
import jax.numpy as jnp
from jax.experimental import pallas as pl


def kernel(x, edge_index, edge_attr, W_lin, att_src, att_dst, W_edge, att_edge, bias, ln_gamma, ln_beta):
    raise NotImplementedError("write your pallas kernel here")


# speedup vs baseline: 17.0418x; 17.0418x over previous
import functools

import jax
import jax.numpy as jnp
from jax import lax
from jax.experimental import pallas as pl
from jax.experimental.pallas import tpu as pltpu
from jax.experimental.pallas import tpu_sc as plsc

_H = 4
_C = 64

_NS = 16
_K = 80
_NROW = 640


def _proj_nodes(x, wb):
    n, d_in = x.shape
    d_out = wb.shape[1]
    blk = 1000

    def body(x_ref, w_ref, o_ref):
        o_ref[...] = jnp.dot(x_ref[...], w_ref[...],
                             preferred_element_type=jnp.float32)

    return pl.pallas_call(
        body,
        grid=(n // blk,),
        in_specs=[
            pl.BlockSpec((blk, d_in), lambda i: (i, 0)),
            pl.BlockSpec((d_in, d_out), lambda i: (0, 0)),
        ],
        out_specs=pl.BlockSpec((blk, d_out), lambda i: (i, 0)),
        out_shape=jax.ShapeDtypeStruct((n, d_out), jnp.float32),
    )(x, wb)


def _proj_edges(edge_attr, v8):
    e, d_in = edge_attr.shape
    d_out = v8.shape[1]
    blk = 2000

    def body(a_ref, v_ref, o_ref):
        o_ref[...] = jnp.dot(a_ref[...], v_ref[...],
                             preferred_element_type=jnp.float32)

    return pl.pallas_call(
        body,
        grid=(e // blk,),
        in_specs=[
            pl.BlockSpec((blk, d_in), lambda i: (i, 0)),
            pl.BlockSpec((d_in, d_out), lambda i: (0, 0)),
        ],
        out_specs=pl.BlockSpec((blk, d_out), lambda i: (i, 0)),
        out_shape=jax.ShapeDtypeStruct((e, d_out), jnp.float32),
    )(edge_attr, v8)


def _finalize(feat, den, x, bias2, gamma2, beta2):
    n, d = x.shape
    blk = 1000

    def body(f_ref, d_ref, x_ref, b_ref, g_ref, be_ref, o_ref):
        pre = f_ref[...] / (d_ref[...] + 1e-16) + b_ref[...] + x_ref[...]
        mu = jnp.mean(pre, axis=1, keepdims=True)
        var = jnp.mean((pre - mu) ** 2, axis=1, keepdims=True)
        y = (pre - mu) / jnp.sqrt(var + 1e-5) * g_ref[...] + be_ref[...]
        o_ref[...] = jnp.where(y > 0, y, jnp.exp(y) - 1.0)

    row = lambda i: (i, 0)
    fixed = lambda i: (0, 0)
    return pl.pallas_call(
        body,
        grid=(n // blk,),
        in_specs=[
            pl.BlockSpec((blk, d), row),
            pl.BlockSpec((blk, d), row),
            pl.BlockSpec((blk, d), row),
            pl.BlockSpec((1, d), fixed),
            pl.BlockSpec((1, d), fixed),
            pl.BlockSpec((1, d), fixed),
        ],
        out_specs=pl.BlockSpec((blk, d), row),
        out_shape=jax.ShapeDtypeStruct((n, d), jnp.float32),
    )(feat, den, x, bias2, gamma2, beta2)


def _sc_aggregate(xlh, tbl, aeP, src3, dst3, n_nodes, n_chunks):
    nrows_t = n_nodes // _NS

    mesh = plsc.VectorSubcoreMesh(core_axis_name="c", subcore_axis_name="s")

    @functools.partial(
        pl.kernel,
        mesh=mesh,
        compiler_params=pltpu.CompilerParams(use_tc_tiling_on_sc=False,
                                             needs_layout_passes=False),
        out_type=jax.ShapeDtypeStruct((2, 2, _NS * _NROW, _K), jnp.float32),
        scratch_types=[
            pltpu.VMEM((n_chunks, _K), jnp.int32),
            pltpu.VMEM((n_chunks, _K), jnp.int32),
            pltpu.VMEM((n_chunks, _K), jnp.float32),
            pltpu.VMEM((n_chunks, _K), jnp.float32),
            pltpu.VMEM((_K, 16), jnp.float32),
            pltpu.VMEM((_K, 16), jnp.float32),
            pltpu.VMEM((_K, 64), jnp.float32),
            pltpu.VMEM((_K, _K), jnp.float32),
            pltpu.VMEM((_K,), jnp.int32),
            pltpu.VMEM((2 * _K,), jnp.float32),
            pltpu.VMEM((2, 16), jnp.float32),
            pltpu.VMEM((_NS, 2, 16), jnp.float32),
            pltpu.VMEM_SHARED((_NS * _NROW, _K), jnp.float32),
            pltpu.VMEM_SHARED((n_nodes, 16), jnp.float32),
            pltpu.VMEM_SHARED((_NS, 2, 16), jnp.float32),
            pltpu.SemaphoreType.DMA,
        ],
    )
    def run(xlh_hbm, tbl_hbm, aeP_hbm, src_hbm, dst_hbm, out_hbm,
            srcv, dstv, al0, al1, tbld, tbls, rows, msg, soffc, aec,
            gbuf, gall, agg_sh, tbl_sh, gmx_sh, gsem):
        c = lax.axis_index("c")
        t = lax.axis_index("s")

        pltpu.sync_copy(src_hbm.at[t], srcv)
        pltpu.sync_copy(dst_hbm.at[t], dstv)
        nb = t * nrows_t
        pltpu.sync_copy(tbl_hbm.at[c, pl.ds(nb, nrows_t)],
                        tbl_sh.at[pl.ds(nb, nrows_t)])

        z16 = jnp.zeros((16,), jnp.float32)
        lane = jnp.arange(16, dtype=jnp.int32)

        def zmsg(i, _):
            msg[i // 5, pl.ds((i % 5) * 16, 16)] = z16
            return 0

        lax.fori_loop(0, _K * 5, zmsg, 0)

        base = t * _NROW

        def zagg(i, _):
            pltpu.sync_copy(msg, agg_sh.at[pl.ds(base + i * _K, _K)])
            return 0

        lax.fori_loop(0, _NROW // _K, zagg, 0)
        plsc.subcore_barrier()

        minit = jnp.full((16,), -3.4e38, jnp.float32)

        def passa(j, carry):
            m0, m1 = carry
            pltpu.sync_copy(aeP_hbm.at[c, t, j], aec)
            cpd = pltpu.async_copy(tbl_sh.at[dstv.at[j]], tbld, gsem)
            cps = pltpu.async_copy(tbl_sh.at[srcv.at[j]], tbls, gsem)
            cpd.wait()
            cps.wait()
            for v in range(5):
                sl = pl.ds(v * 16, 16)
                rw = v * 16 + lane
                si0 = plsc.load_gather(tbld, [rw, jnp.full((16,), 0)])
                si1 = plsc.load_gather(tbld, [rw, jnp.full((16,), 1)])
                sj0 = plsc.load_gather(tbls, [rw, jnp.full((16,), 2)])
                sj1 = plsc.load_gather(tbls, [rw, jnp.full((16,), 3)])
                a0 = si0 + sj0 + aec[sl]
                a0 = jnp.where(a0 >= 0, a0, a0 * 0.2)
                al0[j, sl] = a0
                a1 = si1 + sj1 + aec[pl.ds(_K + v * 16, 16)]
                a1 = jnp.where(a1 >= 0, a1, a1 * 0.2)
                al1[j, sl] = a1
                m0 = jnp.maximum(m0, a0)
                m1 = jnp.maximum(m1, a1)
            return (m0, m1)

        m0, m1 = lax.fori_loop(0, n_chunks, passa, (minit, minit))

        gbuf[0, :] = m0
        gbuf[1, :] = m1
        pltpu.sync_copy(gbuf, gmx_sh.at[t])
        plsc.subcore_barrier()
        pltpu.sync_copy(gmx_sh, gall)

        def redm(i, carry):
            mm0, mm1 = carry
            return (jnp.maximum(mm0, gall[i, 0, :]),
                    jnp.maximum(mm1, gall[i, 1, :]))

        mm0, mm1 = lax.fori_loop(0, _NS, redm, (minit, minit))
        gms = (jnp.max(mm0), jnp.max(mm1))

        for hp in range(2):
            alh = al0 if hp == 0 else al1
            gm = gms[hp]
            hoff = (2 * c + hp) * n_nodes

            def sweep(j, _):
                for v in range(5):
                    sl = pl.ds(v * 16, 16)
                    soffc[sl] = srcv[j, sl] + hoff
                cp = pltpu.async_copy(xlh_hbm.at[soffc], rows, gsem)
                for v in range(5):
                    sl = pl.ds(v * 16, 16)
                    alh[j, sl] = jnp.exp(alh[j, sl] - gm)
                cp.wait()

                def edge_body(e, _):
                    jv = jnp.full((16,), j, jnp.int32)
                    ev = jnp.full((16,), e, jnp.int32)
                    exv = plsc.load_gather(alh, [jv, ev])
                    for s in range(4):
                        sl = pl.ds(s * 16, 16)
                        msg[e, sl] = rows[e, sl] * exv
                    msg[e, pl.ds(64, 16)] = jnp.where(lane == 0, exv, 0.0)
                    return 0

                lax.fori_loop(0, _K, edge_body, 0)
                pltpu.sync_copy(msg, agg_sh.at[dstv.at[j]], add=True)
                return 0

            lax.fori_loop(0, n_chunks, sweep, 0)

            plsc.subcore_barrier()
            pltpu.sync_copy(agg_sh.at[pl.ds(base, _NROW)],
                            out_hbm.at[c, hp, pl.ds(base, _NROW)])
            if hp == 0:
                lax.fori_loop(0, _K * 5, zmsg, 0)
                lax.fori_loop(0, _NROW // _K, zagg, 0)
                plsc.subcore_barrier()

    return run(xlh, tbl, aeP, src3, dst3)


def kernel(x, edge_index, edge_attr, W_lin, att_src, att_dst, W_edge,
           att_edge, bias, ln_gamma, ln_beta):
    n, d_in = x.shape
    e = edge_index.shape[1]
    d_out = W_lin.shape[0]
    e_dim = W_edge.shape[1]
    n_chunks = e // (_NS * _K)

    w_si = jnp.einsum('hcd,hc->dh', W_lin.reshape(_H, _C, d_in), att_src[0])
    w_sj = jnp.einsum('hcd,hc->dh', W_lin.reshape(_H, _C, d_in), att_dst[0])
    wb = jnp.concatenate(
        [W_lin.T, w_si, w_sj, jnp.zeros((d_in, 120), jnp.float32)], axis=1)
    v_e = jnp.einsum('hcd,hc->dh', W_edge.reshape(_H, _C, e_dim), att_edge[0])
    v8 = jnp.concatenate([v_e, jnp.zeros((e_dim, 4), jnp.float32)], axis=1)

    xls = _proj_nodes(x, wb)
    ae8 = _proj_edges(edge_attr, v8)

    xl = xls[:, :d_out]
    si = xls[:, d_out:d_out + 4]
    sj = xls[:, d_out + 4:d_out + 8]
    xlh = xl.reshape(n, _H, _C).transpose(1, 0, 2).reshape(_H * n, _C)
    tbl = jnp.stack([
        jnp.concatenate([si[:, 0:2], sj[:, 0:2],
                         jnp.zeros((n, 12), jnp.float32)], axis=1),
        jnp.concatenate([si[:, 2:4], sj[:, 2:4],
                         jnp.zeros((n, 12), jnp.float32)], axis=1),
    ])
    ae = ae8[:, :4]
    aeP = jnp.stack([
        jnp.concatenate([ae[:, 2 * c].reshape(_NS, n_chunks, _K),
                         ae[:, 2 * c + 1].reshape(_NS, n_chunks, _K)],
                        axis=-1)
        for c in range(2)
    ])
    src3 = edge_index[0].reshape(_NS, n_chunks, _K)
    dst3 = edge_index[1].reshape(_NS, n_chunks, _K)

    aggout = _sc_aggregate(xlh, tbl, aeP, src3, dst3, n, n_chunks)

    feat = jnp.concatenate([aggout[0, 0, :n, :_C], aggout[0, 1, :n, :_C],
                            aggout[1, 0, :n, :_C], aggout[1, 1, :n, :_C]],
                           axis=1)
    den4 = jnp.stack([aggout[0, 0, :n, _C], aggout[0, 1, :n, _C],
                      aggout[1, 0, :n, _C], aggout[1, 1, :n, _C]], axis=1)
    den = jnp.repeat(den4, _C, axis=1)

    return _finalize(feat, den, x, bias.reshape(1, d_out),
                     ln_gamma.reshape(1, d_out), ln_beta.reshape(1, d_out))


# speedup vs baseline: 17.1581x; 1.0068x over previous
import functools

import jax
import jax.numpy as jnp
from jax import lax
from jax.experimental import pallas as pl
from jax.experimental.pallas import tpu as pltpu
from jax.experimental.pallas import tpu_sc as plsc

_H = 4
_C = 64

_NS = 16
_K = 80
_NROW = 640


def _proj_nodes(x, wb):
    n, d_in = x.shape
    d_out = wb.shape[1]
    blk = 1000

    def body(x_ref, w_ref, o_ref):
        o_ref[...] = jnp.dot(x_ref[...], w_ref[...],
                             preferred_element_type=jnp.float32)

    return pl.pallas_call(
        body,
        grid=(n // blk,),
        in_specs=[
            pl.BlockSpec((blk, d_in), lambda i: (i, 0)),
            pl.BlockSpec((d_in, d_out), lambda i: (0, 0)),
        ],
        out_specs=pl.BlockSpec((blk, d_out), lambda i: (i, 0)),
        out_shape=jax.ShapeDtypeStruct((n, d_out), jnp.float32),
    )(x, wb)


def _proj_edges(edge_attr, v8):
    e, d_in = edge_attr.shape
    d_out = v8.shape[1]
    blk = 2000

    def body(a_ref, v_ref, o_ref):
        o_ref[...] = jnp.dot(a_ref[...], v_ref[...],
                             preferred_element_type=jnp.float32)

    return pl.pallas_call(
        body,
        grid=(e // blk,),
        in_specs=[
            pl.BlockSpec((blk, d_in), lambda i: (i, 0)),
            pl.BlockSpec((d_in, d_out), lambda i: (0, 0)),
        ],
        out_specs=pl.BlockSpec((blk, d_out), lambda i: (i, 0)),
        out_shape=jax.ShapeDtypeStruct((e, d_out), jnp.float32),
    )(edge_attr, v8)


def _finalize(feat, den, x, bias2, gamma2, beta2):
    n, d = x.shape
    blk = 1000

    def body(f_ref, d_ref, x_ref, b_ref, g_ref, be_ref, o_ref):
        pre = f_ref[...] / (d_ref[...] + 1e-16) + b_ref[...] + x_ref[...]
        mu = jnp.mean(pre, axis=1, keepdims=True)
        var = jnp.mean((pre - mu) ** 2, axis=1, keepdims=True)
        y = (pre - mu) / jnp.sqrt(var + 1e-5) * g_ref[...] + be_ref[...]
        o_ref[...] = jnp.where(y > 0, y, jnp.exp(y) - 1.0)

    row = lambda i: (i, 0)
    fixed = lambda i: (0, 0)
    return pl.pallas_call(
        body,
        grid=(n // blk,),
        in_specs=[
            pl.BlockSpec((blk, d), row),
            pl.BlockSpec((blk, d), row),
            pl.BlockSpec((blk, d), row),
            pl.BlockSpec((1, d), fixed),
            pl.BlockSpec((1, d), fixed),
            pl.BlockSpec((1, d), fixed),
        ],
        out_specs=pl.BlockSpec((blk, d), row),
        out_shape=jax.ShapeDtypeStruct((n, d), jnp.float32),
    )(feat, den, x, bias2, gamma2, beta2)


def _sc_aggregate(xlh, tbl, aeP, src3, dst3, n_nodes, n_chunks):
    nrows_t = n_nodes // _NS

    mesh = plsc.VectorSubcoreMesh(core_axis_name="c", subcore_axis_name="s")

    @functools.partial(
        pl.kernel,
        mesh=mesh,
        compiler_params=pltpu.CompilerParams(use_tc_tiling_on_sc=False,
                                             needs_layout_passes=False),
        out_type=jax.ShapeDtypeStruct((2, 2, _NS * _NROW, _K), jnp.float32),
        scratch_types=[
            pltpu.VMEM((n_chunks, _K), jnp.int32),
            pltpu.VMEM((n_chunks, _K), jnp.int32),
            pltpu.VMEM((n_chunks, _K), jnp.float32),
            pltpu.VMEM((n_chunks, _K), jnp.float32),
            pltpu.VMEM((_K, 16), jnp.float32),
            pltpu.VMEM((_K, 16), jnp.float32),
            pltpu.VMEM((_K, 64), jnp.float32),
            pltpu.VMEM((_K, _K), jnp.float32),
            pltpu.VMEM((_K,), jnp.int32),
            pltpu.VMEM((2 * _K,), jnp.float32),
            pltpu.VMEM((2, 16), jnp.float32),
            pltpu.VMEM((_NS, 2, 16), jnp.float32),
            pltpu.VMEM_SHARED((_NS * _NROW, _K), jnp.float32),
            pltpu.VMEM_SHARED((n_nodes, 16), jnp.float32),
            pltpu.VMEM_SHARED((_NS, 2, 16), jnp.float32),
            pltpu.SemaphoreType.DMA,
        ],
    )
    def run(xlh_hbm, tbl_hbm, aeP_hbm, src_hbm, dst_hbm, out_hbm,
            srcv, dstv, al0, al1, tbld, tbls, rows, msg, soffc, aec,
            gbuf, gall, agg_sh, tbl_sh, gmx_sh, gsem):
        c = lax.axis_index("c")
        t = lax.axis_index("s")

        pltpu.sync_copy(src_hbm.at[t], srcv)
        pltpu.sync_copy(dst_hbm.at[t], dstv)
        nb = t * nrows_t
        pltpu.sync_copy(tbl_hbm.at[c, pl.ds(nb, nrows_t)],
                        tbl_sh.at[pl.ds(nb, nrows_t)])

        z16 = jnp.zeros((16,), jnp.float32)
        lane = jnp.arange(16, dtype=jnp.int32)

        def zmsg(i, _):
            msg[i // 5, pl.ds((i % 5) * 16, 16)] = z16
            return 0

        lax.fori_loop(0, _K * 5, zmsg, 0)

        base = t * _NROW

        def zagg(i, _):
            pltpu.sync_copy(msg, agg_sh.at[pl.ds(base + i * _K, _K)])
            return 0

        lax.fori_loop(0, _NROW // _K, zagg, 0)
        plsc.subcore_barrier()

        minit = jnp.full((16,), -3.4e38, jnp.float32)

        def passa(j, carry):
            m0, m1 = carry
            pltpu.sync_copy(aeP_hbm.at[c, t, j], aec)
            cpd = pltpu.async_copy(tbl_sh.at[dstv.at[j]], tbld, gsem)
            cps = pltpu.async_copy(tbl_sh.at[srcv.at[j]], tbls, gsem)
            cpd.wait()
            cps.wait()
            for v in range(5):
                sl = pl.ds(v * 16, 16)
                rw = v * 16 + lane
                si0 = plsc.load_gather(tbld, [rw, jnp.full((16,), 0)])
                si1 = plsc.load_gather(tbld, [rw, jnp.full((16,), 1)])
                sj0 = plsc.load_gather(tbls, [rw, jnp.full((16,), 2)])
                sj1 = plsc.load_gather(tbls, [rw, jnp.full((16,), 3)])
                a0 = si0 + sj0 + aec[sl]
                a0 = jnp.where(a0 >= 0, a0, a0 * 0.2)
                al0[j, sl] = a0
                a1 = si1 + sj1 + aec[pl.ds(_K + v * 16, 16)]
                a1 = jnp.where(a1 >= 0, a1, a1 * 0.2)
                al1[j, sl] = a1
                m0 = jnp.maximum(m0, a0)
                m1 = jnp.maximum(m1, a1)
            return (m0, m1)

        m0, m1 = lax.fori_loop(0, n_chunks, passa, (minit, minit))

        gbuf[0, :] = m0
        gbuf[1, :] = m1
        pltpu.sync_copy(gbuf, gmx_sh.at[t])
        plsc.subcore_barrier()
        pltpu.sync_copy(gmx_sh, gall)

        def redm(i, carry):
            mm0, mm1 = carry
            return (jnp.maximum(mm0, gall[i, 0, :]),
                    jnp.maximum(mm1, gall[i, 1, :]))

        mm0, mm1 = lax.fori_loop(0, _NS, redm, (minit, minit))
        gms = (jnp.max(mm0), jnp.max(mm1))

        for hp in range(2):
            alh = al0 if hp == 0 else al1
            gm = gms[hp]
            hoff = (2 * c + hp) * n_nodes

            def sweep(j, _):
                for v in range(5):
                    sl = pl.ds(v * 16, 16)
                    soffc[sl] = srcv[j, sl] + hoff
                cp = pltpu.async_copy(xlh_hbm.at[soffc], rows, gsem)
                for v in range(5):
                    sl = pl.ds(v * 16, 16)
                    alh[j, sl] = jnp.exp(alh[j, sl] - gm)
                cp.wait()

                jv = jnp.full((16,), j, jnp.int32)

                def edge_body(q, _):
                    for u in range(4):
                        e = q * 4 + u
                        ev = jnp.full((16,), e, jnp.int32)
                        exv = plsc.load_gather(alh, [jv, ev])
                        for s in range(4):
                            sl = pl.ds(s * 16, 16)
                            msg[e, sl] = rows[e, sl] * exv
                        msg[e, pl.ds(64, 16)] = jnp.where(lane == 0, exv, 0.0)
                    return 0

                lax.fori_loop(0, _K // 4, edge_body, 0)
                pltpu.sync_copy(msg, agg_sh.at[dstv.at[j]], add=True)
                return 0

            lax.fori_loop(0, n_chunks, sweep, 0)

            plsc.subcore_barrier()
            pltpu.sync_copy(agg_sh.at[pl.ds(base, _NROW)],
                            out_hbm.at[c, hp, pl.ds(base, _NROW)])
            if hp == 0:
                lax.fori_loop(0, _K * 5, zmsg, 0)
                lax.fori_loop(0, _NROW // _K, zagg, 0)
                plsc.subcore_barrier()

    return run(xlh, tbl, aeP, src3, dst3)


def kernel(x, edge_index, edge_attr, W_lin, att_src, att_dst, W_edge,
           att_edge, bias, ln_gamma, ln_beta):
    n, d_in = x.shape
    e = edge_index.shape[1]
    d_out = W_lin.shape[0]
    e_dim = W_edge.shape[1]
    n_chunks = e // (_NS * _K)

    w_si = jnp.einsum('hcd,hc->dh', W_lin.reshape(_H, _C, d_in), att_src[0])
    w_sj = jnp.einsum('hcd,hc->dh', W_lin.reshape(_H, _C, d_in), att_dst[0])
    wb = jnp.concatenate(
        [W_lin.T, w_si, w_sj, jnp.zeros((d_in, 120), jnp.float32)], axis=1)
    v_e = jnp.einsum('hcd,hc->dh', W_edge.reshape(_H, _C, e_dim), att_edge[0])
    v8 = jnp.concatenate([v_e, jnp.zeros((e_dim, 4), jnp.float32)], axis=1)

    xls = _proj_nodes(x, wb)
    ae8 = _proj_edges(edge_attr, v8)

    xl = xls[:, :d_out]
    si = xls[:, d_out:d_out + 4]
    sj = xls[:, d_out + 4:d_out + 8]
    xlh = xl.reshape(n, _H, _C).transpose(1, 0, 2).reshape(_H * n, _C)
    tbl = jnp.stack([
        jnp.concatenate([si[:, 0:2], sj[:, 0:2],
                         jnp.zeros((n, 12), jnp.float32)], axis=1),
        jnp.concatenate([si[:, 2:4], sj[:, 2:4],
                         jnp.zeros((n, 12), jnp.float32)], axis=1),
    ])
    ae = ae8[:, :4]
    aeP = jnp.stack([
        jnp.concatenate([ae[:, 2 * c].reshape(_NS, n_chunks, _K),
                         ae[:, 2 * c + 1].reshape(_NS, n_chunks, _K)],
                        axis=-1)
        for c in range(2)
    ])
    src3 = edge_index[0].reshape(_NS, n_chunks, _K)
    dst3 = edge_index[1].reshape(_NS, n_chunks, _K)

    aggout = _sc_aggregate(xlh, tbl, aeP, src3, dst3, n, n_chunks)

    feat = jnp.concatenate([aggout[0, 0, :n, :_C], aggout[0, 1, :n, :_C],
                            aggout[1, 0, :n, :_C], aggout[1, 1, :n, :_C]],
                           axis=1)
    den4 = jnp.stack([aggout[0, 0, :n, _C], aggout[0, 1, :n, _C],
                      aggout[1, 0, :n, _C], aggout[1, 1, :n, _C]], axis=1)
    den = jnp.repeat(den4, _C, axis=1)

    return _finalize(feat, den, x, bias.reshape(1, d_out),
                     ln_gamma.reshape(1, d_out), ln_beta.reshape(1, d_out))


# speedup vs baseline: 19.1342x; 1.1152x over previous
import functools

import jax
import jax.numpy as jnp
from jax import lax
from jax.experimental import pallas as pl
from jax.experimental.pallas import tpu as pltpu
from jax.experimental.pallas import tpu_sc as plsc

_H = 4
_C = 64

_NS = 16
_K = 80
_NROW = 640


def _proj_nodes(x, wb):
    n, d_in = x.shape
    d_out = wb.shape[1]
    blk = 1000

    def body(x_ref, w_ref, o_ref):
        o_ref[...] = jnp.dot(x_ref[...], w_ref[...],
                             preferred_element_type=jnp.float32)

    return pl.pallas_call(
        body,
        grid=(n // blk,),
        in_specs=[
            pl.BlockSpec((blk, d_in), lambda i: (i, 0)),
            pl.BlockSpec((d_in, d_out), lambda i: (0, 0)),
        ],
        out_specs=pl.BlockSpec((blk, d_out), lambda i: (i, 0)),
        out_shape=jax.ShapeDtypeStruct((n, d_out), jnp.float32),
    )(x, wb)


def _proj_edges(edge_attr, v8):
    e, d_in = edge_attr.shape
    d_out = v8.shape[1]
    blk = 2000

    def body(a_ref, v_ref, o_ref):
        o_ref[...] = jnp.dot(a_ref[...], v_ref[...],
                             preferred_element_type=jnp.float32)

    return pl.pallas_call(
        body,
        grid=(e // blk,),
        in_specs=[
            pl.BlockSpec((blk, d_in), lambda i: (i, 0)),
            pl.BlockSpec((d_in, d_out), lambda i: (0, 0)),
        ],
        out_specs=pl.BlockSpec((blk, d_out), lambda i: (i, 0)),
        out_shape=jax.ShapeDtypeStruct((e, d_out), jnp.float32),
    )(edge_attr, v8)


def _finalize(feat, den, x, bias2, gamma2, beta2):
    n, d = x.shape
    blk = 1000

    def body(f_ref, d_ref, x_ref, b_ref, g_ref, be_ref, o_ref):
        pre = f_ref[...] / (d_ref[...] + 1e-16) + b_ref[...] + x_ref[...]
        mu = jnp.mean(pre, axis=1, keepdims=True)
        var = jnp.mean((pre - mu) ** 2, axis=1, keepdims=True)
        y = (pre - mu) / jnp.sqrt(var + 1e-5) * g_ref[...] + be_ref[...]
        o_ref[...] = jnp.where(y > 0, y, jnp.exp(y) - 1.0)

    row = lambda i: (i, 0)
    fixed = lambda i: (0, 0)
    return pl.pallas_call(
        body,
        grid=(n // blk,),
        in_specs=[
            pl.BlockSpec((blk, d), row),
            pl.BlockSpec((blk, d), row),
            pl.BlockSpec((blk, d), row),
            pl.BlockSpec((1, d), fixed),
            pl.BlockSpec((1, d), fixed),
            pl.BlockSpec((1, d), fixed),
        ],
        out_specs=pl.BlockSpec((blk, d), row),
        out_shape=jax.ShapeDtypeStruct((n, d), jnp.float32),
    )(feat, den, x, bias2, gamma2, beta2)


def _sc_aggregate(xlh, tbl, aeP, src3, dst3, n_nodes, n_chunks):
    nrows_t = n_nodes // _NS

    mesh = plsc.VectorSubcoreMesh(core_axis_name="c", subcore_axis_name="s")

    @functools.partial(
        pl.kernel,
        mesh=mesh,
        compiler_params=pltpu.CompilerParams(use_tc_tiling_on_sc=False,
                                             needs_layout_passes=False),
        out_type=jax.ShapeDtypeStruct((2, 2, _NS * _NROW, _K), jnp.float32),
        scratch_types=[
            pltpu.VMEM((n_chunks, _K), jnp.int32),
            pltpu.VMEM((n_chunks, _K), jnp.int32),
            pltpu.VMEM((n_chunks, _K), jnp.float32),
            pltpu.VMEM((n_chunks, _K), jnp.float32),
            pltpu.VMEM((2, _K, 16), jnp.float32),
            pltpu.VMEM((2, _K, 16), jnp.float32),
            pltpu.VMEM((2, _K, 64), jnp.float32),
            pltpu.VMEM((_K, _K), jnp.float32),
            pltpu.VMEM((2, _K), jnp.int32),
            pltpu.VMEM((2, 2 * _K), jnp.float32),
            pltpu.VMEM((2, 16), jnp.float32),
            pltpu.VMEM((_NS, 2, 16), jnp.float32),
            pltpu.VMEM_SHARED((_NS * _NROW, _K), jnp.float32),
            pltpu.VMEM_SHARED((n_nodes, 16), jnp.float32),
            pltpu.VMEM_SHARED((_NS, 2, 16), jnp.float32),
            pltpu.SemaphoreType.DMA,
            pltpu.SemaphoreType.DMA,
            pltpu.SemaphoreType.DMA,
            pltpu.SemaphoreType.DMA,
        ],
    )
    def run(xlh_hbm, tbl_hbm, aeP_hbm, src_hbm, dst_hbm, out_hbm,
            srcv, dstv, al0, al1, tbld, tbls, rows, msg, soffc, aec,
            gbuf, gall, agg_sh, tbl_sh, gmx_sh, sem0, sem1, sem2, sem3):
        c = lax.axis_index("c")
        t = lax.axis_index("s")

        pltpu.sync_copy(src_hbm.at[t], srcv)
        pltpu.sync_copy(dst_hbm.at[t], dstv)
        nb = t * nrows_t
        pltpu.sync_copy(tbl_hbm.at[c, pl.ds(nb, nrows_t)],
                        tbl_sh.at[pl.ds(nb, nrows_t)])

        z16 = jnp.zeros((16,), jnp.float32)
        lane = jnp.arange(16, dtype=jnp.int32)

        def zmsg(i, _):
            msg[i // 5, pl.ds((i % 5) * 16, 16)] = z16
            return 0

        lax.fori_loop(0, _K * 5, zmsg, 0)

        base = t * _NROW

        def zagg(i, _):
            pltpu.sync_copy(msg, agg_sh.at[pl.ds(base + i * _K, _K)])
            return 0

        lax.fori_loop(0, _NROW // _K, zagg, 0)
        plsc.subcore_barrier()

        minit = jnp.full((16,), -3.4e38, jnp.float32)
        sems = (sem0, sem1)

        lsems = (sem2, sem3)

        def issue_a(j, b):
            return [
                pltpu.async_copy(aeP_hbm.at[c, t, j], aec.at[b], lsems[b]),
                pltpu.async_copy(tbl_sh.at[dstv.at[j]], tbld.at[b], sems[b]),
                pltpu.async_copy(tbl_sh.at[srcv.at[j]], tbls.at[b], sems[b]),
            ]

        def compute_a(j, b, m0, m1):
            for v in range(5):
                sl = pl.ds(v * 16, 16)
                rw = v * 16 + lane
                si0 = plsc.load_gather(tbld.at[b], [rw, jnp.full((16,), 0)])
                si1 = plsc.load_gather(tbld.at[b], [rw, jnp.full((16,), 1)])
                sj0 = plsc.load_gather(tbls.at[b], [rw, jnp.full((16,), 2)])
                sj1 = plsc.load_gather(tbls.at[b], [rw, jnp.full((16,), 3)])
                a0 = si0 + sj0 + aec[b, sl]
                a0 = jnp.where(a0 >= 0, a0, a0 * 0.2)
                al0[j, sl] = a0
                a1 = si1 + sj1 + aec[b, pl.ds(_K + v * 16, 16)]
                a1 = jnp.where(a1 >= 0, a1, a1 * 0.2)
                al1[j, sl] = a1
                m0 = jnp.maximum(m0, a0)
                m1 = jnp.maximum(m1, a1)
            return m0, m1

        def passa(gp, carry):
            m0, m1 = carry
            j0 = gp * 2
            da = issue_a(j0, 0)
            db = issue_a(j0 + 1, 1)
            for d in da:
                d.wait()
            m0, m1 = compute_a(j0, 0, m0, m1)
            for d in db:
                d.wait()
            m0, m1 = compute_a(j0 + 1, 1, m0, m1)
            return (m0, m1)

        m0, m1 = lax.fori_loop(0, n_chunks // 2, passa, (minit, minit))
        jt = n_chunks - 1
        for d in issue_a(jt, 0):
            d.wait()
        m0, m1 = compute_a(jt, 0, m0, m1)

        gbuf[0, :] = m0
        gbuf[1, :] = m1
        pltpu.sync_copy(gbuf, gmx_sh.at[t])
        plsc.subcore_barrier()
        pltpu.sync_copy(gmx_sh, gall)

        def redm(i, carry):
            mm0, mm1 = carry
            return (jnp.maximum(mm0, gall[i, 0, :]),
                    jnp.maximum(mm1, gall[i, 1, :]))

        mm0, mm1 = lax.fori_loop(0, _NS, redm, (minit, minit))
        gms = (jnp.max(mm0), jnp.max(mm1))

        for hp in range(2):
            alh = al0 if hp == 0 else al1
            gm = gms[hp]
            hoff = (2 * c + hp) * n_nodes

            def issue_s(j, b):
                for v in range(5):
                    sl = pl.ds(v * 16, 16)
                    soffc[b, sl] = srcv[j, sl] + hoff
                return pltpu.async_copy(xlh_hbm.at[soffc.at[b]], rows.at[b],
                                        sems[b])

            def compute_s(j, b, cp):
                for v in range(5):
                    sl = pl.ds(v * 16, 16)
                    alh[j, sl] = jnp.exp(alh[j, sl] - gm)
                cp.wait()
                jv = jnp.full((16,), j, jnp.int32)

                def edge_body(q, _):
                    for u in range(4):
                        e = q * 4 + u
                        ev = jnp.full((16,), e, jnp.int32)
                        exv = plsc.load_gather(alh, [jv, ev])
                        for s in range(4):
                            sl = pl.ds(s * 16, 16)
                            msg[e, sl] = rows[b, e, sl] * exv
                        msg[e, pl.ds(64, 16)] = jnp.where(lane == 0, exv, 0.0)
                    return 0

                lax.fori_loop(0, _K // 4, edge_body, 0)
                pltpu.sync_copy(msg, agg_sh.at[dstv.at[j]], add=True)

            def sweep(gp, _):
                j0 = gp * 2
                c0 = issue_s(j0, 0)
                c1 = issue_s(j0 + 1, 1)
                compute_s(j0, 0, c0)
                compute_s(j0 + 1, 1, c1)
                return 0

            lax.fori_loop(0, n_chunks // 2, sweep, 0)
            jt = n_chunks - 1
            compute_s(jt, 0, issue_s(jt, 0))

            plsc.subcore_barrier()
            pltpu.sync_copy(agg_sh.at[pl.ds(base, _NROW)],
                            out_hbm.at[c, hp, pl.ds(base, _NROW)])
            if hp == 0:
                lax.fori_loop(0, _K * 5, zmsg, 0)
                lax.fori_loop(0, _NROW // _K, zagg, 0)
                plsc.subcore_barrier()

    return run(xlh, tbl, aeP, src3, dst3)


def kernel(x, edge_index, edge_attr, W_lin, att_src, att_dst, W_edge,
           att_edge, bias, ln_gamma, ln_beta):
    n, d_in = x.shape
    e = edge_index.shape[1]
    d_out = W_lin.shape[0]
    e_dim = W_edge.shape[1]
    n_chunks = e // (_NS * _K)

    w_si = jnp.einsum('hcd,hc->dh', W_lin.reshape(_H, _C, d_in), att_src[0])
    w_sj = jnp.einsum('hcd,hc->dh', W_lin.reshape(_H, _C, d_in), att_dst[0])
    wb = jnp.concatenate(
        [W_lin.T, w_si, w_sj, jnp.zeros((d_in, 120), jnp.float32)], axis=1)
    v_e = jnp.einsum('hcd,hc->dh', W_edge.reshape(_H, _C, e_dim), att_edge[0])
    v8 = jnp.concatenate([v_e, jnp.zeros((e_dim, 4), jnp.float32)], axis=1)

    xls = _proj_nodes(x, wb)
    ae8 = _proj_edges(edge_attr, v8)

    xl = xls[:, :d_out]
    si = xls[:, d_out:d_out + 4]
    sj = xls[:, d_out + 4:d_out + 8]
    xlh = xl.reshape(n, _H, _C).transpose(1, 0, 2).reshape(_H * n, _C)
    tbl = jnp.stack([
        jnp.concatenate([si[:, 0:2], sj[:, 0:2],
                         jnp.zeros((n, 12), jnp.float32)], axis=1),
        jnp.concatenate([si[:, 2:4], sj[:, 2:4],
                         jnp.zeros((n, 12), jnp.float32)], axis=1),
    ])
    ae = ae8[:, :4]
    aeP = jnp.stack([
        jnp.concatenate([ae[:, 2 * c].reshape(_NS, n_chunks, _K),
                         ae[:, 2 * c + 1].reshape(_NS, n_chunks, _K)],
                        axis=-1)
        for c in range(2)
    ])
    src3 = edge_index[0].reshape(_NS, n_chunks, _K)
    dst3 = edge_index[1].reshape(_NS, n_chunks, _K)

    aggout = _sc_aggregate(xlh, tbl, aeP, src3, dst3, n, n_chunks)

    feat = jnp.concatenate([aggout[0, 0, :n, :_C], aggout[0, 1, :n, :_C],
                            aggout[1, 0, :n, :_C], aggout[1, 1, :n, :_C]],
                           axis=1)
    den4 = jnp.stack([aggout[0, 0, :n, _C], aggout[0, 1, :n, _C],
                      aggout[1, 0, :n, _C], aggout[1, 1, :n, _C]], axis=1)
    den = jnp.repeat(den4, _C, axis=1)

    return _finalize(feat, den, x, bias.reshape(1, d_out),
                     ln_gamma.reshape(1, d_out), ln_beta.reshape(1, d_out))


# speedup vs baseline: 22.4708x; 1.1744x over previous
import functools

import jax
import jax.numpy as jnp
from jax import lax
from jax.experimental import pallas as pl
from jax.experimental.pallas import tpu as pltpu
from jax.experimental.pallas import tpu_sc as plsc

_H = 4
_C = 64

_NS = 16
_K = 80
_NROW = 640


def _proj_nodes(x, wb5):
    n, d_in = x.shape
    blk = 2000

    def body(x_ref, w_ref, o_ref):
        o_ref[0] = jnp.dot(x_ref[...], w_ref[0],
                           preferred_element_type=jnp.float32)

    return pl.pallas_call(
        body,
        grid=(n // blk, 5),
        in_specs=[
            pl.BlockSpec((blk, d_in), lambda i, j: (i, 0)),
            pl.BlockSpec((1, d_in, 64), lambda i, j: (j, 0, 0)),
        ],
        out_specs=pl.BlockSpec((1, blk, 64), lambda i, j: (j, i, 0)),
        out_shape=jax.ShapeDtypeStruct((5, n, 64), jnp.float32),
    )(x, wb5)


def _proj_edges(edge_attr, v8):
    e, d_in = edge_attr.shape
    d_out = v8.shape[1]
    blk = 2000

    def body(a_ref, v_ref, o_ref):
        o_ref[...] = jnp.dot(a_ref[...], v_ref[...],
                             preferred_element_type=jnp.float32)

    return pl.pallas_call(
        body,
        grid=(e // blk,),
        in_specs=[
            pl.BlockSpec((blk, d_in), lambda i: (i, 0)),
            pl.BlockSpec((d_in, d_out), lambda i: (0, 0)),
        ],
        out_specs=pl.BlockSpec((blk, d_out), lambda i: (i, 0)),
        out_shape=jax.ShapeDtypeStruct((e, d_out), jnp.float32),
    )(edge_attr, v8)


def _finalize(aggs, x, bias2, gamma2, beta2):
    n, d = x.shape
    blk = 1000

    def body(a0_ref, a1_ref, a2_ref, a3_ref, x_ref, b_ref, g_ref, be_ref,
             o_ref):
        parts = []
        for a_ref in (a0_ref, a1_ref, a2_ref, a3_ref):
            a = a_ref[...]
            den = jnp.broadcast_to(a[:, 64:65], (blk, _C))
            parts.append(a[:, :_C] / (den + 1e-16))
        pre = jnp.concatenate(parts, axis=1) + b_ref[...] + x_ref[...]
        mu = jnp.mean(pre, axis=1, keepdims=True)
        var = jnp.mean((pre - mu) ** 2, axis=1, keepdims=True)
        y = (pre - mu) / jnp.sqrt(var + 1e-5) * g_ref[...] + be_ref[...]
        o_ref[...] = jnp.where(y > 0, y, jnp.exp(y) - 1.0)

    arow = lambda i: (i, 0)
    fixed = lambda i: (0, 0)
    return pl.pallas_call(
        body,
        grid=(n // blk,),
        in_specs=[
            pl.BlockSpec((blk, _K), arow),
            pl.BlockSpec((blk, _K), arow),
            pl.BlockSpec((blk, _K), arow),
            pl.BlockSpec((blk, _K), arow),
            pl.BlockSpec((blk, d), arow),
            pl.BlockSpec((1, d), fixed),
            pl.BlockSpec((1, d), fixed),
            pl.BlockSpec((1, d), fixed),
        ],
        out_specs=pl.BlockSpec((blk, d), arow),
        out_shape=jax.ShapeDtypeStruct((n, d), jnp.float32),
    )(aggs[0], aggs[1], aggs[2], aggs[3], x, bias2, gamma2, beta2)


def _sc_aggregate(xlh, tbl, ae4, src3, dst3, n_nodes, n_chunks):
    nrows_t = n_nodes // _NS

    mesh = plsc.VectorSubcoreMesh(core_axis_name="c", subcore_axis_name="s")

    @functools.partial(
        pl.kernel,
        mesh=mesh,
        compiler_params=pltpu.CompilerParams(use_tc_tiling_on_sc=False,
                                             needs_layout_passes=False),
        out_type=jax.ShapeDtypeStruct((2, 2, _NS * _NROW, _K), jnp.float32),
        scratch_types=[
            pltpu.VMEM((n_chunks, _K), jnp.int32),
            pltpu.VMEM((n_chunks, _K), jnp.int32),
            pltpu.VMEM((n_chunks, _K), jnp.float32),
            pltpu.VMEM((n_chunks, _K), jnp.float32),
            pltpu.VMEM((2, _K, 16), jnp.float32),
            pltpu.VMEM((2, _K, 16), jnp.float32),
            pltpu.VMEM((2, _K, 64), jnp.float32),
            pltpu.VMEM((_K, _K), jnp.float32),
            pltpu.VMEM((2, _K), jnp.int32),
            pltpu.VMEM((2, _K, 8), jnp.float32),
            pltpu.VMEM((2, 16), jnp.float32),
            pltpu.VMEM((_NS, 2, 16), jnp.float32),
            pltpu.VMEM_SHARED((_NS * _NROW, _K), jnp.float32),
            pltpu.VMEM_SHARED((n_nodes, 16), jnp.float32),
            pltpu.VMEM_SHARED((_NS, 2, 16), jnp.float32),
            pltpu.SemaphoreType.DMA,
            pltpu.SemaphoreType.DMA,
            pltpu.SemaphoreType.DMA,
            pltpu.SemaphoreType.DMA,
        ],
    )
    def run(xlh_hbm, tbl_hbm, ae4_hbm, src_hbm, dst_hbm, out_hbm,
            srcv, dstv, al0, al1, tbld, tbls, rows, msg, soffc, aec,
            gbuf, gall, agg_sh, tbl_sh, gmx_sh, sem0, sem1, sem2, sem3):
        c = lax.axis_index("c")
        t = lax.axis_index("s")

        pltpu.sync_copy(src_hbm.at[t], srcv)
        pltpu.sync_copy(dst_hbm.at[t], dstv)
        nb = t * nrows_t
        pltpu.sync_copy(tbl_hbm.at[pl.ds(nb, nrows_t), pl.ds(16 * c, 16)],
                        tbl_sh.at[pl.ds(nb, nrows_t)])

        z16 = jnp.zeros((16,), jnp.float32)
        lane = jnp.arange(16, dtype=jnp.int32)

        def zmsg(i, _):
            msg[i // 5, pl.ds((i % 5) * 16, 16)] = z16
            return 0

        lax.fori_loop(0, _K * 5, zmsg, 0)

        base = t * _NROW

        def zagg(i, _):
            pltpu.sync_copy(msg, agg_sh.at[pl.ds(base + i * _K, _K)])
            return 0

        lax.fori_loop(0, _NROW // _K, zagg, 0)
        plsc.subcore_barrier()

        minit = jnp.full((16,), -3.4e38, jnp.float32)
        sems = (sem0, sem1)

        lsems = (sem2, sem3)

        def issue_a(j, b):
            return [
                pltpu.async_copy(ae4_hbm.at[t, j], aec.at[b], lsems[b]),
                pltpu.async_copy(tbl_sh.at[dstv.at[j]], tbld.at[b], sems[b]),
                pltpu.async_copy(tbl_sh.at[srcv.at[j]], tbls.at[b], sems[b]),
            ]

        h0v = 2 * c + jnp.zeros((16,), jnp.int32)
        h1v = h0v + 1

        def compute_a(j, b, m0, m1):
            for v in range(5):
                sl = pl.ds(v * 16, 16)
                rw = v * 16 + lane
                si0 = plsc.load_gather(tbld.at[b], [rw, jnp.full((16,), 0)])
                si1 = plsc.load_gather(tbld.at[b], [rw, jnp.full((16,), 1)])
                sj0 = plsc.load_gather(tbls.at[b], [rw, jnp.full((16,), 2)])
                sj1 = plsc.load_gather(tbls.at[b], [rw, jnp.full((16,), 3)])
                ae0 = plsc.load_gather(aec.at[b], [rw, h0v])
                ae1 = plsc.load_gather(aec.at[b], [rw, h1v])
                a0 = si0 + sj0 + ae0
                a0 = jnp.where(a0 >= 0, a0, a0 * 0.2)
                al0[j, sl] = a0
                a1 = si1 + sj1 + ae1
                a1 = jnp.where(a1 >= 0, a1, a1 * 0.2)
                al1[j, sl] = a1
                m0 = jnp.maximum(m0, a0)
                m1 = jnp.maximum(m1, a1)
            return m0, m1

        def passa(gp, carry):
            m0, m1 = carry
            j0 = gp * 2
            da = issue_a(j0, 0)
            db = issue_a(j0 + 1, 1)
            for d in da:
                d.wait()
            m0, m1 = compute_a(j0, 0, m0, m1)
            for d in db:
                d.wait()
            m0, m1 = compute_a(j0 + 1, 1, m0, m1)
            return (m0, m1)

        m0, m1 = lax.fori_loop(0, n_chunks // 2, passa, (minit, minit))
        jt = n_chunks - 1
        for d in issue_a(jt, 0):
            d.wait()
        m0, m1 = compute_a(jt, 0, m0, m1)

        gbuf[0, :] = m0
        gbuf[1, :] = m1
        pltpu.sync_copy(gbuf, gmx_sh.at[t])
        plsc.subcore_barrier()
        pltpu.sync_copy(gmx_sh, gall)

        def redm(i, carry):
            mm0, mm1 = carry
            return (jnp.maximum(mm0, gall[i, 0, :]),
                    jnp.maximum(mm1, gall[i, 1, :]))

        mm0, mm1 = lax.fori_loop(0, _NS, redm, (minit, minit))
        gms = (jnp.max(mm0), jnp.max(mm1))

        for hp in range(2):
            alh = al0 if hp == 0 else al1
            gm = gms[hp]
            hoff = (2 * c + hp) * n_nodes

            def issue_s(j, b):
                for v in range(5):
                    sl = pl.ds(v * 16, 16)
                    soffc[b, sl] = srcv[j, sl] + hoff
                return pltpu.async_copy(xlh_hbm.at[soffc.at[b]], rows.at[b],
                                        sems[b])

            def compute_s(j, b, cp):
                for v in range(5):
                    sl = pl.ds(v * 16, 16)
                    alh[j, sl] = jnp.exp(alh[j, sl] - gm)
                cp.wait()
                jv = jnp.full((16,), j, jnp.int32)

                def edge_body(q, _):
                    for u in range(4):
                        e = q * 4 + u
                        ev = jnp.full((16,), e, jnp.int32)
                        exv = plsc.load_gather(alh, [jv, ev])
                        for s in range(4):
                            sl = pl.ds(s * 16, 16)
                            msg[e, sl] = rows[b, e, sl] * exv
                        msg[e, pl.ds(64, 16)] = jnp.where(lane == 0, exv, 0.0)
                    return 0

                lax.fori_loop(0, _K // 4, edge_body, 0)
                pltpu.sync_copy(msg, agg_sh.at[dstv.at[j]], add=True)

            def sweep(gp, _):
                j0 = gp * 2
                c0 = issue_s(j0, 0)
                c1 = issue_s(j0 + 1, 1)
                compute_s(j0, 0, c0)
                compute_s(j0 + 1, 1, c1)
                return 0

            lax.fori_loop(0, n_chunks // 2, sweep, 0)
            jt = n_chunks - 1
            compute_s(jt, 0, issue_s(jt, 0))

            plsc.subcore_barrier()
            pltpu.sync_copy(agg_sh.at[pl.ds(base, _NROW)],
                            out_hbm.at[c, hp, pl.ds(base, _NROW)])
            if hp == 0:
                lax.fori_loop(0, _K * 5, zmsg, 0)
                lax.fori_loop(0, _NROW // _K, zagg, 0)
                plsc.subcore_barrier()

    return run(xlh, tbl, ae4, src3, dst3)


def kernel(x, edge_index, edge_attr, W_lin, att_src, att_dst, W_edge,
           att_edge, bias, ln_gamma, ln_beta):
    n, d_in = x.shape
    e = edge_index.shape[1]
    d_out = W_lin.shape[0]
    e_dim = W_edge.shape[1]
    n_chunks = e // (_NS * _K)

    w_si = jnp.einsum('hcd,hc->dh', W_lin.reshape(_H, _C, d_in), att_src[0])
    w_sj = jnp.einsum('hcd,hc->dh', W_lin.reshape(_H, _C, d_in), att_dst[0])
    z12 = jnp.zeros((d_in, 12), jnp.float32)
    blk4 = jnp.concatenate(
        [w_si[:, 0:2], w_sj[:, 0:2], z12,
         w_si[:, 2:4], w_sj[:, 2:4], z12,
         jnp.zeros((d_in, 32), jnp.float32)], axis=1)
    wb = jnp.concatenate([W_lin.T, blk4], axis=1)
    wb5 = wb.reshape(d_in, 5, 64).transpose(1, 0, 2)
    v_e = jnp.einsum('hcd,hc->dh', W_edge.reshape(_H, _C, e_dim), att_edge[0])
    v8 = jnp.concatenate([v_e, jnp.zeros((e_dim, 4), jnp.float32)], axis=1)

    out5 = _proj_nodes(x, wb5)
    ae8 = _proj_edges(edge_attr, v8)

    xlh = out5[:4].reshape(_H * n, _C)
    tbl = out5[4]
    ae4 = ae8.reshape(_NS, n_chunks, _K, 8)
    src3 = edge_index[0].reshape(_NS, n_chunks, _K)
    dst3 = edge_index[1].reshape(_NS, n_chunks, _K)

    aggout = _sc_aggregate(xlh, tbl, ae4, src3, dst3, n, n_chunks)

    aggs = (aggout[0, 0, :n], aggout[0, 1, :n],
            aggout[1, 0, :n], aggout[1, 1, :n])
    return _finalize(aggs, x, bias.reshape(1, d_out),
                     ln_gamma.reshape(1, d_out), ln_beta.reshape(1, d_out))


# speedup vs baseline: 24.4210x; 1.0868x over previous
import functools

import jax
import jax.numpy as jnp
from jax import lax
from jax.experimental import pallas as pl
from jax.experimental.pallas import tpu as pltpu
from jax.experimental.pallas import tpu_sc as plsc

_H = 4
_C = 64

_NS = 16
_K = 80
_NROW = 640


def _proj_nodes(x, wb5):
    n, d_in = x.shape
    blk = 2000

    def body(x_ref, w_ref, o_ref):
        xb = x_ref[...]
        for h in range(5):
            o_ref[h] = jnp.dot(xb, w_ref[h],
                               preferred_element_type=jnp.float32)

    return pl.pallas_call(
        body,
        grid=(n // blk,),
        in_specs=[
            pl.BlockSpec((blk, d_in), lambda i: (i, 0)),
            pl.BlockSpec((5, d_in, 64), lambda i: (0, 0, 0)),
        ],
        out_specs=pl.BlockSpec((5, blk, 64), lambda i: (0, i, 0)),
        out_shape=jax.ShapeDtypeStruct((5, n, 64), jnp.float32),
    )(x, wb5)


def _proj_edges(edge_attr, v8):
    e, d_in = edge_attr.shape
    d_out = v8.shape[1]
    blk = 16000

    def body(a_ref, v_ref, o_ref):
        o_ref[...] = jnp.dot(a_ref[...], v_ref[...],
                             preferred_element_type=jnp.float32)

    return pl.pallas_call(
        body,
        grid=(e // blk,),
        in_specs=[
            pl.BlockSpec((blk, d_in), lambda i: (i, 0)),
            pl.BlockSpec((d_in, d_out), lambda i: (0, 0)),
        ],
        out_specs=pl.BlockSpec((blk, d_out), lambda i: (i, 0)),
        out_shape=jax.ShapeDtypeStruct((e, d_out), jnp.float32),
    )(edge_attr, v8)


def _finalize(aggs, x, bias2, gamma2, beta2):
    n, d = x.shape
    blk = 1000

    def body(a0_ref, a1_ref, a2_ref, a3_ref, x_ref, b_ref, g_ref, be_ref,
             o_ref):
        parts = []
        for a_ref in (a0_ref, a1_ref, a2_ref, a3_ref):
            a = a_ref[0, 0]
            den = jnp.broadcast_to(a[:, 64:65], (blk, _C))
            parts.append(a[:, :_C] / (den + 1e-16))
        pre = jnp.concatenate(parts, axis=1) + b_ref[...] + x_ref[...]
        mu = jnp.mean(pre, axis=1, keepdims=True)
        var = jnp.mean((pre - mu) ** 2, axis=1, keepdims=True)
        y = (pre - mu) / jnp.sqrt(var + 1e-5) * g_ref[...] + be_ref[...]
        o_ref[...] = jnp.where(y > 0, y, jnp.exp(y) - 1.0)

    arow = lambda i: (i, 0)
    fixed = lambda i: (0, 0)
    aspec = lambda c, hp: pl.BlockSpec((1, 1, blk, _K),
                                       lambda i, c=c, hp=hp: (c, hp, i, 0))
    return pl.pallas_call(
        body,
        grid=(n // blk,),
        in_specs=[
            aspec(0, 0),
            aspec(0, 1),
            aspec(1, 0),
            aspec(1, 1),
            pl.BlockSpec((blk, d), arow),
            pl.BlockSpec((1, d), fixed),
            pl.BlockSpec((1, d), fixed),
            pl.BlockSpec((1, d), fixed),
        ],
        out_specs=pl.BlockSpec((blk, d), arow),
        out_shape=jax.ShapeDtypeStruct((n, d), jnp.float32),
    )(aggs, aggs, aggs, aggs, x, bias2, gamma2, beta2)


def _sc_aggregate(xlh, ae4, src3, dst3, n_nodes, n_chunks):
    nrows_t = n_nodes // _NS

    mesh = plsc.VectorSubcoreMesh(core_axis_name="c", subcore_axis_name="s")

    @functools.partial(
        pl.kernel,
        mesh=mesh,
        compiler_params=pltpu.CompilerParams(use_tc_tiling_on_sc=False,
                                             needs_layout_passes=False),
        out_type=jax.ShapeDtypeStruct((2, 2, _NS * _NROW, _K), jnp.float32),
        scratch_types=[
            pltpu.VMEM((n_chunks, _K), jnp.int32),
            pltpu.VMEM((n_chunks, _K), jnp.int32),
            pltpu.VMEM((n_chunks, _K), jnp.float32),
            pltpu.VMEM((n_chunks, _K), jnp.float32),
            pltpu.VMEM((2, _K, 16), jnp.float32),
            pltpu.VMEM((2, _K, 16), jnp.float32),
            pltpu.VMEM((2, _K, 64), jnp.float32),
            pltpu.VMEM((_K, _K), jnp.float32),
            pltpu.VMEM((2, _K), jnp.int32),
            pltpu.VMEM((2, _K, 8), jnp.float32),
            pltpu.VMEM((2, 16), jnp.float32),
            pltpu.VMEM((_NS, 2, 16), jnp.float32),
            pltpu.VMEM_SHARED((_NS * _NROW, _K), jnp.float32),
            pltpu.VMEM_SHARED((n_nodes, 16), jnp.float32),
            pltpu.VMEM_SHARED((_NS, 2, 16), jnp.float32),
            pltpu.SemaphoreType.DMA,
            pltpu.SemaphoreType.DMA,
            pltpu.SemaphoreType.DMA,
            pltpu.SemaphoreType.DMA,
        ],
    )
    def run(xlh_hbm, ae4_hbm, src_hbm, dst_hbm, out_hbm,
            srcv, dstv, al0, al1, tbld, tbls, rows, msg, soffc, aec,
            gbuf, gall, agg_sh, tbl_sh, gmx_sh, sem0, sem1, sem2, sem3):
        c = lax.axis_index("c")
        t = lax.axis_index("s")

        pltpu.sync_copy(src_hbm.at[t], srcv)
        pltpu.sync_copy(dst_hbm.at[t], dstv)
        nb = t * nrows_t
        pltpu.sync_copy(
            xlh_hbm.at[pl.ds(4 * n_nodes + nb, nrows_t), pl.ds(16 * c, 16)],
            tbl_sh.at[pl.ds(nb, nrows_t)])

        z16 = jnp.zeros((16,), jnp.float32)
        lane = jnp.arange(16, dtype=jnp.int32)

        def zmsg(i, _):
            msg[i // 5, pl.ds((i % 5) * 16, 16)] = z16
            return 0

        lax.fori_loop(0, _K * 5, zmsg, 0)

        base = t * _NROW

        def zagg(i, _):
            pltpu.sync_copy(msg, agg_sh.at[pl.ds(base + i * _K, _K)])
            return 0

        lax.fori_loop(0, _NROW // _K, zagg, 0)
        plsc.subcore_barrier()

        minit = jnp.full((16,), -3.4e38, jnp.float32)
        sems = (sem0, sem1)

        lsems = (sem2, sem3)

        def issue_a(j, b):
            return [
                pltpu.async_copy(ae4_hbm.at[t, j], aec.at[b], lsems[b]),
                pltpu.async_copy(tbl_sh.at[dstv.at[j]], tbld.at[b], sems[b]),
                pltpu.async_copy(tbl_sh.at[srcv.at[j]], tbls.at[b], sems[b]),
            ]

        h0v = 2 * c + jnp.zeros((16,), jnp.int32)
        h1v = h0v + 1

        def compute_a(j, b, m0, m1):
            for v in range(5):
                sl = pl.ds(v * 16, 16)
                rw = v * 16 + lane
                si0 = plsc.load_gather(tbld.at[b], [rw, jnp.full((16,), 0)])
                si1 = plsc.load_gather(tbld.at[b], [rw, jnp.full((16,), 1)])
                sj0 = plsc.load_gather(tbls.at[b], [rw, jnp.full((16,), 2)])
                sj1 = plsc.load_gather(tbls.at[b], [rw, jnp.full((16,), 3)])
                ae0 = plsc.load_gather(aec.at[b], [rw, h0v])
                ae1 = plsc.load_gather(aec.at[b], [rw, h1v])
                a0 = si0 + sj0 + ae0
                a0 = jnp.where(a0 >= 0, a0, a0 * 0.2)
                al0[j, sl] = a0
                a1 = si1 + sj1 + ae1
                a1 = jnp.where(a1 >= 0, a1, a1 * 0.2)
                al1[j, sl] = a1
                m0 = jnp.maximum(m0, a0)
                m1 = jnp.maximum(m1, a1)
            return m0, m1

        def passa(gp, carry):
            m0, m1 = carry
            j0 = gp * 2
            da = issue_a(j0, 0)
            db = issue_a(j0 + 1, 1)
            for d in da:
                d.wait()
            m0, m1 = compute_a(j0, 0, m0, m1)
            for d in db:
                d.wait()
            m0, m1 = compute_a(j0 + 1, 1, m0, m1)
            return (m0, m1)

        m0, m1 = lax.fori_loop(0, n_chunks // 2, passa, (minit, minit))
        jt = n_chunks - 1
        for d in issue_a(jt, 0):
            d.wait()
        m0, m1 = compute_a(jt, 0, m0, m1)

        gbuf[0, :] = m0
        gbuf[1, :] = m1
        pltpu.sync_copy(gbuf, gmx_sh.at[t])
        plsc.subcore_barrier()
        pltpu.sync_copy(gmx_sh, gall)

        def redm(i, carry):
            mm0, mm1 = carry
            return (jnp.maximum(mm0, gall[i, 0, :]),
                    jnp.maximum(mm1, gall[i, 1, :]))

        mm0, mm1 = lax.fori_loop(0, _NS, redm, (minit, minit))
        gms = (jnp.max(mm0), jnp.max(mm1))

        for hp in range(2):
            alh = al0 if hp == 0 else al1
            gm = gms[hp]
            hoff = (2 * c + hp) * n_nodes

            def issue_s(j, b):
                for v in range(5):
                    sl = pl.ds(v * 16, 16)
                    soffc[b, sl] = srcv[j, sl] + hoff
                return pltpu.async_copy(xlh_hbm.at[soffc.at[b]], rows.at[b],
                                        sems[b])

            def compute_s(j, b, cp):
                for v in range(5):
                    sl = pl.ds(v * 16, 16)
                    alh[j, sl] = jnp.exp(alh[j, sl] - gm)
                cp.wait()
                jv = jnp.full((16,), j, jnp.int32)

                def edge_body(q, _):
                    for u in range(4):
                        e = q * 4 + u
                        ev = jnp.full((16,), e, jnp.int32)
                        exv = plsc.load_gather(alh, [jv, ev])
                        for s in range(4):
                            sl = pl.ds(s * 16, 16)
                            msg[e, sl] = rows[b, e, sl] * exv
                        msg[e, pl.ds(64, 16)] = jnp.where(lane == 0, exv, 0.0)
                    return 0

                lax.fori_loop(0, _K // 4, edge_body, 0)
                pltpu.sync_copy(msg, agg_sh.at[dstv.at[j]], add=True)

            def sweep(gp, _):
                j0 = gp * 2
                c0 = issue_s(j0, 0)
                c1 = issue_s(j0 + 1, 1)
                compute_s(j0, 0, c0)
                compute_s(j0 + 1, 1, c1)
                return 0

            lax.fori_loop(0, n_chunks // 2, sweep, 0)
            jt = n_chunks - 1
            compute_s(jt, 0, issue_s(jt, 0))

            plsc.subcore_barrier()
            pltpu.sync_copy(agg_sh.at[pl.ds(base, _NROW)],
                            out_hbm.at[c, hp, pl.ds(base, _NROW)])
            if hp == 0:
                lax.fori_loop(0, _K * 5, zmsg, 0)
                lax.fori_loop(0, _NROW // _K, zagg, 0)
                plsc.subcore_barrier()

    return run(xlh, ae4, src3, dst3)


def kernel(x, edge_index, edge_attr, W_lin, att_src, att_dst, W_edge,
           att_edge, bias, ln_gamma, ln_beta):
    n, d_in = x.shape
    e = edge_index.shape[1]
    d_out = W_lin.shape[0]
    e_dim = W_edge.shape[1]
    n_chunks = e // (_NS * _K)

    w_si = jnp.einsum('hcd,hc->dh', W_lin.reshape(_H, _C, d_in), att_src[0])
    w_sj = jnp.einsum('hcd,hc->dh', W_lin.reshape(_H, _C, d_in), att_dst[0])
    z12 = jnp.zeros((d_in, 12), jnp.float32)
    blk4 = jnp.concatenate(
        [w_si[:, 0:2], w_sj[:, 0:2], z12,
         w_si[:, 2:4], w_sj[:, 2:4], z12,
         jnp.zeros((d_in, 32), jnp.float32)], axis=1)
    wb = jnp.concatenate([W_lin.T, blk4], axis=1)
    wb5 = wb.reshape(d_in, 5, 64).transpose(1, 0, 2)
    v_e = jnp.einsum('hcd,hc->dh', W_edge.reshape(_H, _C, e_dim), att_edge[0])
    v8 = jnp.concatenate([v_e, jnp.zeros((e_dim, 4), jnp.float32)], axis=1)

    out5 = _proj_nodes(x, wb5)
    ae8 = _proj_edges(edge_attr, v8)

    xlh = out5.reshape(5 * n, _C)
    ae4 = ae8.reshape(_NS, n_chunks, _K, 8)
    src3 = edge_index[0].reshape(_NS, n_chunks, _K)
    dst3 = edge_index[1].reshape(_NS, n_chunks, _K)

    aggout = _sc_aggregate(xlh, ae4, src3, dst3, n, n_chunks)

    return _finalize(aggout, x, bias.reshape(1, d_out),
                     ln_gamma.reshape(1, d_out), ln_beta.reshape(1, d_out))


# speedup vs baseline: 24.4214x; 1.0000x over previous
import functools

import jax
import jax.numpy as jnp
from jax import lax
from jax.experimental import pallas as pl
from jax.experimental.pallas import tpu as pltpu
from jax.experimental.pallas import tpu_sc as plsc

_H = 4
_C = 64

_NS = 16
_K = 80
_NROW = 640


def _proj_nodes(x, wb5):
    n, d_in = x.shape
    blk = 2000

    def body(x_ref, w_ref, o_ref):
        xb = x_ref[...]
        for h in range(5):
            o_ref[h] = jnp.dot(xb, w_ref[h],
                               preferred_element_type=jnp.float32)

    return pl.pallas_call(
        body,
        grid=(n // blk,),
        in_specs=[
            pl.BlockSpec((blk, d_in), lambda i: (i, 0)),
            pl.BlockSpec((5, d_in, 64), lambda i: (0, 0, 0)),
        ],
        out_specs=pl.BlockSpec((5, blk, 64), lambda i: (0, i, 0)),
        out_shape=jax.ShapeDtypeStruct((5, n, 64), jnp.float32),
    )(x, wb5)


def _proj_edges(edge_attr, v8):
    e, d_in = edge_attr.shape
    d_out = v8.shape[1]
    blk = 16000

    def body(a_ref, v_ref, o_ref):
        o_ref[...] = jnp.dot(a_ref[...], v_ref[...],
                             preferred_element_type=jnp.float32)

    return pl.pallas_call(
        body,
        grid=(e // blk,),
        in_specs=[
            pl.BlockSpec((blk, d_in), lambda i: (i, 0)),
            pl.BlockSpec((d_in, d_out), lambda i: (0, 0)),
        ],
        out_specs=pl.BlockSpec((blk, d_out), lambda i: (i, 0)),
        out_shape=jax.ShapeDtypeStruct((e, d_out), jnp.float32),
    )(edge_attr, v8)


def _finalize(aggs, x, bias2, gamma2, beta2):
    n, d = x.shape
    blk = 1000

    def body(a0_ref, a1_ref, a2_ref, a3_ref, x_ref, b_ref, g_ref, be_ref,
             o_ref):
        parts = []
        for a_ref in (a0_ref, a1_ref, a2_ref, a3_ref):
            a = a_ref[0, 0]
            den = jnp.broadcast_to(a[:, 64:65], (blk, _C))
            parts.append(a[:, :_C] / (den + 1e-16))
        pre = jnp.concatenate(parts, axis=1) + b_ref[...] + x_ref[...]
        mu = jnp.mean(pre, axis=1, keepdims=True)
        var = jnp.mean((pre - mu) ** 2, axis=1, keepdims=True)
        y = (pre - mu) / jnp.sqrt(var + 1e-5) * g_ref[...] + be_ref[...]
        o_ref[...] = jnp.where(y > 0, y, jnp.exp(y) - 1.0)

    arow = lambda i: (i, 0)
    fixed = lambda i: (0, 0)
    aspec = lambda c, hp: pl.BlockSpec((1, 1, blk, _K),
                                       lambda i, c=c, hp=hp: (c, hp, i, 0))
    return pl.pallas_call(
        body,
        grid=(n // blk,),
        in_specs=[
            aspec(0, 0),
            aspec(0, 1),
            aspec(1, 0),
            aspec(1, 1),
            pl.BlockSpec((blk, d), arow),
            pl.BlockSpec((1, d), fixed),
            pl.BlockSpec((1, d), fixed),
            pl.BlockSpec((1, d), fixed),
        ],
        out_specs=pl.BlockSpec((blk, d), arow),
        out_shape=jax.ShapeDtypeStruct((n, d), jnp.float32),
    )(aggs, aggs, aggs, aggs, x, bias2, gamma2, beta2)


def _sc_aggregate(xlh, ae4, src3, dst3, n_nodes, n_chunks):
    nrows_t = n_nodes // _NS

    mesh = plsc.VectorSubcoreMesh(core_axis_name="c", subcore_axis_name="s")

    @functools.partial(
        pl.kernel,
        mesh=mesh,
        compiler_params=pltpu.CompilerParams(use_tc_tiling_on_sc=False,
                                             needs_layout_passes=False),
        out_type=jax.ShapeDtypeStruct((2, 2, _NS * _NROW, _K), jnp.float32),
        scratch_types=[
            pltpu.VMEM((n_chunks, _K), jnp.int32),
            pltpu.VMEM((n_chunks, _K), jnp.int32),
            pltpu.VMEM((n_chunks, _K), jnp.float32),
            pltpu.VMEM((n_chunks, _K), jnp.float32),
            pltpu.VMEM((2, _K, 16), jnp.float32),
            pltpu.VMEM((2, _K, 16), jnp.float32),
            pltpu.VMEM((2, _K, 64), jnp.float32),
            pltpu.VMEM((_K, _K), jnp.float32),
            pltpu.VMEM((2, _K), jnp.int32),
            pltpu.VMEM((2, _K, 8), jnp.float32),
            pltpu.VMEM((2, 16), jnp.float32),
            pltpu.VMEM((_NS, 2, 16), jnp.float32),
            pltpu.VMEM_SHARED((_NS * _NROW, _K), jnp.float32),
            pltpu.VMEM_SHARED((n_nodes, 16), jnp.float32),
            pltpu.VMEM_SHARED((_NS, 2, 16), jnp.float32),
            [pltpu.SemaphoreType.DMA] * 2,
            [pltpu.SemaphoreType.DMA] * 2,
        ],
    )
    def run(xlh_hbm, ae4_hbm, src_hbm, dst_hbm, out_hbm,
            srcv, dstv, al0, al1, tbld, tbls, rows, msg, soffc, aec,
            gbuf, gall, agg_sh, tbl_sh, gmx_sh, sems, lsems):
        c = lax.axis_index("c")
        t = lax.axis_index("s")

        pltpu.sync_copy(src_hbm.at[t], srcv)
        pltpu.sync_copy(dst_hbm.at[t], dstv)
        nb = t * nrows_t
        pltpu.sync_copy(
            xlh_hbm.at[pl.ds(4 * n_nodes + nb, nrows_t), pl.ds(16 * c, 16)],
            tbl_sh.at[pl.ds(nb, nrows_t)])

        z16 = jnp.zeros((16,), jnp.float32)
        lane = jnp.arange(16, dtype=jnp.int32)

        def zmsg(i, _):
            msg[i // 5, pl.ds((i % 5) * 16, 16)] = z16
            return 0

        lax.fori_loop(0, _K * 5, zmsg, 0)

        base = t * _NROW

        def zagg(i, _):
            pltpu.sync_copy(msg, agg_sh.at[pl.ds(base + i * _K, _K)])
            return 0

        lax.fori_loop(0, _NROW // _K, zagg, 0)
        plsc.subcore_barrier()

        minit = jnp.full((16,), -3.4e38, jnp.float32)

        def issue_a(j, b):
            return [
                pltpu.async_copy(ae4_hbm.at[t, j], aec.at[b], lsems[b]),
                pltpu.async_copy(tbl_sh.at[dstv.at[j]], tbld.at[b], sems[b]),
                pltpu.async_copy(tbl_sh.at[srcv.at[j]], tbls.at[b], sems[b]),
            ]

        h0v = 2 * c + jnp.zeros((16,), jnp.int32)
        h1v = h0v + 1

        def compute_a(j, b, m0, m1):
            for v in range(5):
                sl = pl.ds(v * 16, 16)
                rw = v * 16 + lane
                si0 = plsc.load_gather(tbld.at[b], [rw, jnp.full((16,), 0)])
                si1 = plsc.load_gather(tbld.at[b], [rw, jnp.full((16,), 1)])
                sj0 = plsc.load_gather(tbls.at[b], [rw, jnp.full((16,), 2)])
                sj1 = plsc.load_gather(tbls.at[b], [rw, jnp.full((16,), 3)])
                ae0 = plsc.load_gather(aec.at[b], [rw, h0v])
                ae1 = plsc.load_gather(aec.at[b], [rw, h1v])
                a0 = si0 + sj0 + ae0
                a0 = jnp.where(a0 >= 0, a0, a0 * 0.2)
                al0[j, sl] = a0
                a1 = si1 + sj1 + ae1
                a1 = jnp.where(a1 >= 0, a1, a1 * 0.2)
                al1[j, sl] = a1
                m0 = jnp.maximum(m0, a0)
                m1 = jnp.maximum(m1, a1)
            return m0, m1

        def passa(gp, carry):
            m0, m1 = carry
            j0 = gp * 2
            descs = [issue_a(j0 + b, b) for b in range(2)]
            for b in range(2):
                for d in descs[b]:
                    d.wait()
                m0, m1 = compute_a(j0 + b, b, m0, m1)
            return (m0, m1)

        m0, m1 = lax.fori_loop(0, n_chunks // 2, passa, (minit, minit))
        jt = n_chunks - 1
        for d in issue_a(jt, 0):
            d.wait()
        m0, m1 = compute_a(jt, 0, m0, m1)

        gbuf[0, :] = m0
        gbuf[1, :] = m1
        pltpu.sync_copy(gbuf, gmx_sh.at[t])
        plsc.subcore_barrier()
        pltpu.sync_copy(gmx_sh, gall)

        def redm(i, carry):
            mm0, mm1 = carry
            return (jnp.maximum(mm0, gall[i, 0, :]),
                    jnp.maximum(mm1, gall[i, 1, :]))

        mm0, mm1 = lax.fori_loop(0, _NS, redm, (minit, minit))
        gms = (jnp.max(mm0), jnp.max(mm1))

        for hp in range(2):
            alh = al0 if hp == 0 else al1
            gm = gms[hp]
            hoff = (2 * c + hp) * n_nodes

            def issue_s(j, b):
                for v in range(5):
                    sl = pl.ds(v * 16, 16)
                    soffc[b, sl] = srcv[j, sl] + hoff
                return pltpu.async_copy(xlh_hbm.at[soffc.at[b]], rows.at[b],
                                        sems[b])

            def compute_s(j, b, cp):
                for v in range(5):
                    sl = pl.ds(v * 16, 16)
                    alh[j, sl] = jnp.exp(alh[j, sl] - gm)
                cp.wait()
                jv = jnp.full((16,), j, jnp.int32)

                def edge_body(q, _):
                    for u in range(4):
                        e = q * 4 + u
                        ev = jnp.full((16,), e, jnp.int32)
                        exv = plsc.load_gather(alh, [jv, ev])
                        for s in range(4):
                            sl = pl.ds(s * 16, 16)
                            msg[e, sl] = rows[b, e, sl] * exv
                        msg[e, pl.ds(64, 16)] = jnp.where(lane == 0, exv, 0.0)
                    return 0

                lax.fori_loop(0, _K // 4, edge_body, 0)
                pltpu.sync_copy(msg, agg_sh.at[dstv.at[j]], add=True)

            def sweep(gp, _):
                j0 = gp * 2
                descs = [issue_s(j0 + b, b) for b in range(2)]
                for b in range(2):
                    compute_s(j0 + b, b, descs[b])
                return 0

            lax.fori_loop(0, n_chunks // 2, sweep, 0)
            jt = n_chunks - 1
            compute_s(jt, 0, issue_s(jt, 0))

            plsc.subcore_barrier()
            pltpu.sync_copy(agg_sh.at[pl.ds(base, _NROW)],
                            out_hbm.at[c, hp, pl.ds(base, _NROW)])
            if hp == 0:
                lax.fori_loop(0, _K * 5, zmsg, 0)
                lax.fori_loop(0, _NROW // _K, zagg, 0)
                plsc.subcore_barrier()

    return run(xlh, ae4, src3, dst3)


def kernel(x, edge_index, edge_attr, W_lin, att_src, att_dst, W_edge,
           att_edge, bias, ln_gamma, ln_beta):
    n, d_in = x.shape
    e = edge_index.shape[1]
    d_out = W_lin.shape[0]
    e_dim = W_edge.shape[1]
    n_chunks = e // (_NS * _K)

    w_si = jnp.einsum('hcd,hc->dh', W_lin.reshape(_H, _C, d_in), att_src[0])
    w_sj = jnp.einsum('hcd,hc->dh', W_lin.reshape(_H, _C, d_in), att_dst[0])
    z12 = jnp.zeros((d_in, 12), jnp.float32)
    blk4 = jnp.concatenate(
        [w_si[:, 0:2], w_sj[:, 0:2], z12,
         w_si[:, 2:4], w_sj[:, 2:4], z12,
         jnp.zeros((d_in, 32), jnp.float32)], axis=1)
    wb = jnp.concatenate([W_lin.T, blk4], axis=1)
    wb5 = wb.reshape(d_in, 5, 64).transpose(1, 0, 2)
    v_e = jnp.einsum('hcd,hc->dh', W_edge.reshape(_H, _C, e_dim), att_edge[0])
    v8 = jnp.concatenate([v_e, jnp.zeros((e_dim, 4), jnp.float32)], axis=1)

    out5 = _proj_nodes(x, wb5)
    ae8 = _proj_edges(edge_attr, v8)

    xlh = out5.reshape(5 * n, _C)
    ae4 = ae8.reshape(_NS, n_chunks, _K, 8)
    src3 = edge_index[0].reshape(_NS, n_chunks, _K)
    dst3 = edge_index[1].reshape(_NS, n_chunks, _K)

    aggout = _sc_aggregate(xlh, ae4, src3, dst3, n, n_chunks)

    return _finalize(aggout, x, bias.reshape(1, d_out),
                     ln_gamma.reshape(1, d_out), ln_beta.reshape(1, d_out))
